# baseline (speedup 1.0000x reference)
import jax
import jax.numpy as jnp
from jax import lax
from jax.experimental import pallas as pl
from jax.experimental.pallas import tpu as pltpu
from jax.experimental.pallas import tpu_sc as plsc

BATCH = 16384
EMB = 64
FEAT = 128
HID = 64
VPAIR = 500000

NC = 2
NS = 16
NW = NC * NS
B_PER_W = BATCH // NW
CHUNK = 128
NCHUNK = B_PER_W // CHUNK


def _gather_body(table_hbm, idx_hbm, out_hbm, idx_v, rows_v, sem):
    wid = lax.axis_index("s") * NC + lax.axis_index("c")
    base = wid * B_PER_W
    pltpu.sync_copy(idx_hbm.at[wid], idx_v)
    copies = []
    for j in range(NCHUNK):
        copies.append(
            pltpu.async_copy(
                table_hbm.at[idx_v.at[j]],
                rows_v.at[pl.ds(j * CHUNK, CHUNK)],
                sem,
            )
        )
    for c in copies:
        c.wait()
    pltpu.sync_copy(rows_v, out_hbm.at[pl.ds(base, B_PER_W)])


_sc_gather = pl.kernel(
    _gather_body,
    mesh=plsc.VectorSubcoreMesh(core_axis_name="c", subcore_axis_name="s"),
    out_type=jax.ShapeDtypeStruct((BATCH, 128), jnp.float32),
    scratch_types=[
        pltpu.VMEM((NCHUNK, CHUNK), jnp.int32),
        pltpu.VMEM((B_PER_W, 128), jnp.float32),
        pltpu.SemaphoreType.DMA,
    ],
)


BM = 2048


def _mm_body(feat_ref, w_ref, b_ref, out_ref):
    out_ref[...] = lax.dot_general(
        feat_ref[...],
        w_ref[...],
        (((1,), (1,)), ((), ())),
        preferred_element_type=jnp.float32,
    ) + b_ref[...]


def _matmul(feat, W, b2):
    return pl.pallas_call(
        _mm_body,
        grid=(BATCH // BM,),
        in_specs=[
            pl.BlockSpec((BM, FEAT), lambda i: (i, 0)),
            pl.BlockSpec((HID, FEAT), lambda i: (0, 0)),
            pl.BlockSpec((1, HID), lambda i: (0, 0)),
        ],
        out_specs=pl.BlockSpec((BM, HID), lambda i: (i, 0)),
        out_shape=jax.ShapeDtypeStruct((BATCH, HID), jnp.float32),
    )(feat, W, b2)


def _cat_body(left_ref, gath_ref, par_ref, out_ref):
    rows = gath_ref[...]
    emb = jnp.where(par_ref[...] > 0, rows[:, EMB:], rows[:, :EMB])
    out_ref[...] = jnp.concatenate([left_ref[...], emb], axis=-1)


def _concat(left, gath, par):
    return pl.pallas_call(
        _cat_body,
        grid=(BATCH // BM,),
        in_specs=[
            pl.BlockSpec((BM, HID), lambda i: (i, 0)),
            pl.BlockSpec((BM, 128), lambda i: (i, 0)),
            pl.BlockSpec((BM, 1), lambda i: (i, 0)),
        ],
        out_specs=pl.BlockSpec((BM, HID + EMB), lambda i: (i, 0)),
        out_shape=jax.ShapeDtypeStruct((BATCH, HID + EMB), jnp.float32),
    )(left, gath, par)


def kernel(feat, id, W, b, table):
    ids = jnp.minimum(id.astype(jnp.int32), 2 * VPAIR - 1)
    pair_idx = (ids // 2).reshape(NW, NCHUNK, CHUNK)
    par = (ids % 2).reshape(BATCH, 1)
    one = jnp.float32(1.0) + W[0, 0] * jnp.float32(1e-45)
    table2 = (table[: 2 * VPAIR] * one).reshape(VPAIR, 128)
    left = _matmul(feat, W, b.reshape(1, HID))
    gath = _sc_gather(table2, pair_idx)
    return _concat(left, gath, par)


# speedup vs baseline: 1.2469x; 1.2469x over previous
import jax
import jax.numpy as jnp
from jax import lax
from jax.experimental import pallas as pl
from jax.experimental.pallas import tpu as pltpu
from jax.experimental.pallas import tpu_sc as plsc

BATCH = 16384
EMB = 64
FEAT = 128
HID = 64
VPAIR = 500000

NC = 2
NS = 16
NW = NC * NS
B_PER_W = BATCH // NW
CHUNK = 128
NCHUNK = B_PER_W // CHUNK


def _gather_body(table_hbm, idx_hbm, out_hbm, idx_v, rows_v, sem):
    wid = lax.axis_index("s") * NC + lax.axis_index("c")
    base = wid * B_PER_W
    pltpu.sync_copy(idx_hbm.at[wid], idx_v)
    copies = []
    for j in range(NCHUNK):
        copies.append(
            pltpu.async_copy(
                table_hbm.at[idx_v.at[j]],
                rows_v.at[pl.ds(j * CHUNK, CHUNK)],
                sem,
            )
        )
    for c in copies:
        c.wait()
    pltpu.sync_copy(rows_v, out_hbm.at[pl.ds(base, B_PER_W)])


_sc_gather = pl.kernel(
    _gather_body,
    mesh=plsc.VectorSubcoreMesh(core_axis_name="c", subcore_axis_name="s"),
    out_type=jax.ShapeDtypeStruct((BATCH, 128), jnp.float32),
    scratch_types=[
        pltpu.VMEM((NCHUNK, CHUNK), jnp.int32),
        pltpu.VMEM((B_PER_W, 128), jnp.float32),
        pltpu.SemaphoreType.DMA,
    ],
)


BM = 2048


def _mm_body(feat_ref, w_ref, b_ref, out_ref):
    out_ref[...] = lax.dot_general(
        feat_ref[...],
        w_ref[...],
        (((1,), (1,)), ((), ())),
        preferred_element_type=jnp.float32,
    ) + b_ref[...]


def _matmul(feat, W, b2):
    return pl.pallas_call(
        _mm_body,
        grid=(BATCH // BM,),
        in_specs=[
            pl.BlockSpec((BM, FEAT), lambda i: (i, 0)),
            pl.BlockSpec((HID, FEAT), lambda i: (0, 0)),
            pl.BlockSpec((1, HID), lambda i: (0, 0)),
        ],
        out_specs=pl.BlockSpec((BM, HID), lambda i: (i, 0)),
        out_shape=jax.ShapeDtypeStruct((BATCH, HID), jnp.float32),
    )(feat, W, b2)


def _cat_body(left_ref, gath_ref, par_ref, out_ref):
    rows = gath_ref[...]
    emb = jnp.where(par_ref[...] > 0, rows[:, EMB:], rows[:, :EMB])
    out_ref[...] = jnp.concatenate([left_ref[...], emb], axis=-1)


def _concat(left, gath, par):
    return pl.pallas_call(
        _cat_body,
        grid=(BATCH // BM,),
        in_specs=[
            pl.BlockSpec((BM, HID), lambda i: (i, 0)),
            pl.BlockSpec((BM, 128), lambda i: (i, 0)),
            pl.BlockSpec((BM, 1), lambda i: (i, 0)),
        ],
        out_specs=pl.BlockSpec((BM, HID + EMB), lambda i: (i, 0)),
        out_shape=jax.ShapeDtypeStruct((BATCH, HID + EMB), jnp.float32),
    )(left, gath, par)


def kernel(feat, id, W, b, table):
    ids = jnp.minimum(id.astype(jnp.int32), 2 * VPAIR - 1)
    pair_idx = (ids // 2).reshape(NW, NCHUNK, CHUNK)
    par = (ids % 2).reshape(BATCH, 1)
    table2 = table[: 2 * VPAIR].reshape(VPAIR, 128)
    left = _matmul(feat, W, b.reshape(1, HID))
    gath = _sc_gather(table2, pair_idx)
    return _concat(left, gath, par)
